# baseline (speedup 1.0000x reference)
import numpy as np
import jax
import jax.numpy as jnp
from jax.experimental import pallas as pl
from jax.experimental.pallas import tpu as pltpu

_N = 10000
_D = 128
_BM = 200
_GRID = _N // _BM
_BC = 2048
_NC = -(-_N // _BC)
_NPAD = _NC * _BC
_GS = 4
_RG = _GS * _BM
_NG = -(-_GRID // _GS)

_HI = jax.lax.Precision.HIGHEST


def _lrelu(v):
    return jnp.where(v >= 0.0, v, 0.01 * v)


def _fused_kernel(kof_ref, jof_ref, adj_ref, blk_ref, xbf_ref, w1_ref, b1_ref,
                  w2_ref, b2_ref, x_ref, wm1_ref, bm1_ref, wm2_ref, bm2_ref,
                  watt_ref, batt_ref, g1_ref, g2_ref, mlp_ref, gk_ref,
                  s2_ref, acc_ref):
    t = pl.program_id(0)

    @pl.when(t == 0)
    def _init_pad():
        s2_ref[pl.ds(_N, _NPAD - _N), :] = jnp.zeros(
            (_NPAD - _N, _D), jnp.bfloat16)

    @pl.when(t < _GRID)
    def _phase1():
        k = jnp.minimum(t, _GRID - 1)
        row = k * _BM
        a = adj_ref[...].astype(jnp.bfloat16)
        u = jax.lax.dot(a, xbf_ref[...], preferred_element_type=jnp.float32)
        g1 = _lrelu(jax.lax.dot(u, w1_ref[...], precision=_HI,
                                preferred_element_type=jnp.float32) + b1_ref[...])
        g1_ref[...] = g1
        s2 = jax.lax.dot(g1, w2_ref[...], precision=_HI,
                         preferred_element_type=jnp.float32)
        s2_ref[pl.ds(row, _BM), :] = s2.astype(jnp.bfloat16)
        nr = (_RG * (k // _GS)) // _BC
        acc_ref[pl.ds(row, _BM), :] = jnp.zeros((_BM, _D), jnp.float32)
        for j2 in range(_N // _BC):
            @pl.when(j2 < nr)
            def _consume():
                part = jax.lax.dot(
                    a[:, j2 * _BC:(j2 + 1) * _BC],
                    s2_ref[pl.ds(j2 * _BC, _BC), :],
                    preferred_element_type=jnp.float32)
                acc_ref[pl.ds(row, _BM), :] += part

    @pl.when(t >= _GRID)
    def _phase2():
        g = kof_ref[t]
        j2 = jof_ref[t]
        grow = g * _RG
        a = blk_ref[...].astype(jnp.bfloat16)

        @pl.when(j2 < _NC - 1)
        def _acc_full():
            part = jax.lax.dot(a, s2_ref[pl.ds(j2 * _BC, _BC), :],
                               preferred_element_type=jnp.float32)
            acc_ref[pl.ds(grow, _RG), :] += part

        @pl.when(j2 == _NC - 1)
        def _acc_last():
            lane = jax.lax.broadcasted_iota(jnp.int32, (_RG, _BC), 1)
            am = jnp.where(lane < _N - (_NC - 1) * _BC, a, jnp.bfloat16(0))
            part = jax.lax.dot(am, s2_ref[pl.ds(j2 * _BC, _BC), :],
                               preferred_element_type=jnp.float32)
            acc_ref[pl.ds(grow, _RG), :] += part

        @pl.when(j2 == _NC - 1)
        def _epilogue():
            g2 = _lrelu(acc_ref[pl.ds(grow, _RG), :] + b2_ref[...])
            g2_ref[...] = g2
            x = x_ref[...]
            h = jax.lax.dot(x, wm1_ref[...], precision=_HI,
                            preferred_element_type=jnp.float32) + bm1_ref[...]
            h = jnp.maximum(h, 0.0)
            mlp = jax.lax.dot(h, wm2_ref[...], precision=_HI,
                              preferred_element_type=jnp.float32) + bm2_ref[...]
            mlp_ref[...] = mlp
            w = watt_ref[...]
            b = batt_ref[0, 0]
            sg = jnp.sum(g2 * w, axis=1, keepdims=True) + b
            sm = jnp.sum(mlp * w, axis=1, keepdims=True) + b
            m = jnp.maximum(sg, sm)
            eg = jnp.exp(sg - m)
            em = jnp.exp(sm - m)
            ag = eg / (eg + em)
            gk_ref[...] = ag * g2 + (1.0 - ag) * mlp


def _schedule():
    kof, jof = [0] * _GRID, [0] * _GRID
    for g in range(_NG):
        nr = (_RG * g) // _BC
        for j2 in range(nr, _NC):
            kof.append(g)
            jof.append(j2)
    return np.asarray(kof, np.int32), np.asarray(jof, np.int32)


def _full_spec(shape):
    return pl.BlockSpec(shape, lambda t, kof, jof: (0,) * len(shape))


def kernel(x, adj, W1, b1, W2, b2, Wm1, bm1, Wm2, bm2, w_att, b_att):
    b1r = b1.reshape(1, _D)
    b2r = b2.reshape(1, _D)
    bm1r = bm1.reshape(1, _D)
    bm2r = bm2.reshape(1, _D)
    wattr = w_att.reshape(1, _D)
    battr = b_att.reshape(1, 1)
    x_bf = x.astype(jnp.bfloat16)

    kof, jof = _schedule()
    nsteps = kof.shape[0]

    adj_spec = pl.BlockSpec(
        (_BM, _N), lambda t, ko, jo: (jnp.minimum(t, _GRID - 1), 0))
    blk_spec = pl.BlockSpec((_RG, _BC), lambda t, ko, jo: (ko[t], jo[t]))
    p1_rows = pl.BlockSpec(
        (_BM, _D), lambda t, ko, jo: (jnp.minimum(t, _GRID - 1), 0))
    p2_rows = pl.BlockSpec((_RG, _D), lambda t, ko, jo: (ko[t], 0))

    grid_spec = pltpu.PrefetchScalarGridSpec(
        num_scalar_prefetch=2,
        grid=(nsteps,),
        in_specs=[
            adj_spec,
            blk_spec,
            _full_spec((_N, _D)),
            _full_spec((_D, _D)),
            _full_spec((1, _D)),
            _full_spec((_D, _D)),
            _full_spec((1, _D)),
            p2_rows,
            _full_spec((_D, _D)),
            _full_spec((1, _D)),
            _full_spec((_D, _D)),
            _full_spec((1, _D)),
            _full_spec((1, _D)),
            _full_spec((1, 1)),
        ],
        out_specs=(p1_rows, p2_rows, p2_rows, p2_rows),
        scratch_shapes=[
            pltpu.VMEM((_NPAD, _D), jnp.bfloat16),
            pltpu.VMEM((_NG * _RG, _D), jnp.float32),
        ],
    )

    g_1, g_2, mlp_feat, gk = pl.pallas_call(
        _fused_kernel,
        grid_spec=grid_spec,
        out_shape=(
            jax.ShapeDtypeStruct((_N, _D), jnp.float32),
            jax.ShapeDtypeStruct((_N, _D), jnp.float32),
            jax.ShapeDtypeStruct((_N, _D), jnp.float32),
            jax.ShapeDtypeStruct((_N, _D), jnp.float32),
        ),
    )(jnp.asarray(kof), jnp.asarray(jof), adj, adj, x_bf, W1, b1r, W2, b2r,
      x, Wm1, bm1r, Wm2, bm2r, wattr, battr)

    return (g_1, g_2, mlp_feat, gk)


# speedup vs baseline: 1.1134x; 1.1134x over previous
import jax
import jax.numpy as jnp
from jax.experimental import pallas as pl
from jax.experimental.pallas import tpu as pltpu

_N = 10000
_D = 128
_BM = 560
_GRID = -(-_N // _BM)
_SPAD = _GRID * _BM

_HI = jax.lax.Precision.HIGHEST


def _lrelu(v):
    return jnp.where(v >= 0.0, v, 0.01 * v)


def _fused_kernel(adj_ref, xbf_ref, w1_ref, b1_ref, w2_ref, b2_ref, x_ref,
                  wm1_ref, bm1_ref, wm2_ref, bm2_ref, watt_ref, batt_ref,
                  g1_ref, g2_ref, mlp_ref, gk_ref, s2_ref):
    i = pl.program_id(0)

    @pl.when(i < _GRID)
    def _pass1():
        a = adj_ref[...].astype(jnp.bfloat16)
        u = jax.lax.dot(a, xbf_ref[...], preferred_element_type=jnp.float32)
        g1 = _lrelu(jax.lax.dot(u, w1_ref[...], precision=_HI,
                                preferred_element_type=jnp.float32) + b1_ref[...])
        g1_ref[...] = g1
        s2 = jax.lax.dot(g1, w2_ref[...], precision=_HI,
                         preferred_element_type=jnp.float32)
        row = jnp.minimum(i, _GRID - 1) * _BM
        s2_ref[pl.ds(row, _BM), :] = s2.astype(jnp.bfloat16)

    @pl.when(i >= _GRID)
    def _pass2():
        a = adj_ref[...].astype(jnp.bfloat16)
        acc = jax.lax.dot(a, s2_ref[:_N, :],
                          preferred_element_type=jnp.float32)
        g2 = _lrelu(acc + b2_ref[...])
        g2_ref[...] = g2
        x = x_ref[...]
        h = jax.lax.dot(x, wm1_ref[...], precision=_HI,
                        preferred_element_type=jnp.float32) + bm1_ref[...]
        h = jnp.maximum(h, 0.0)
        mlp = jax.lax.dot(h, wm2_ref[...], precision=_HI,
                          preferred_element_type=jnp.float32) + bm2_ref[...]
        mlp_ref[...] = mlp
        w = watt_ref[...]
        b = batt_ref[0, 0]
        sg = jnp.sum(g2 * w, axis=1, keepdims=True) + b
        sm = jnp.sum(mlp * w, axis=1, keepdims=True) + b
        m = jnp.maximum(sg, sm)
        eg = jnp.exp(sg - m)
        em = jnp.exp(sm - m)
        ag = eg / (eg + em)
        gk_ref[...] = ag * g2 + (1.0 - ag) * mlp


def _full_spec(shape):
    return pl.BlockSpec(shape, lambda i: (0,) * len(shape))


def kernel(x, adj, W1, b1, W2, b2, Wm1, bm1, Wm2, bm2, w_att, b_att):
    b1r = b1.reshape(1, _D)
    b2r = b2.reshape(1, _D)
    bm1r = bm1.reshape(1, _D)
    bm2r = bm2.reshape(1, _D)
    wattr = w_att.reshape(1, _D)
    battr = b_att.reshape(1, 1)
    x_bf = x.astype(jnp.bfloat16)

    adj_spec = pl.BlockSpec((_BM, _N), lambda i: (i % _GRID, 0))
    p1_rows = pl.BlockSpec((_BM, _D), lambda i: (jnp.minimum(i, _GRID - 1), 0))
    p2_rows = pl.BlockSpec((_BM, _D), lambda i: (jnp.maximum(i - _GRID, 0), 0))

    g_1, g_2, mlp_feat, gk = pl.pallas_call(
        _fused_kernel,
        grid=(2 * _GRID,),
        in_specs=[
            adj_spec,
            _full_spec((_N, _D)),
            _full_spec((_D, _D)),
            _full_spec((1, _D)),
            _full_spec((_D, _D)),
            _full_spec((1, _D)),
            p2_rows,
            _full_spec((_D, _D)),
            _full_spec((1, _D)),
            _full_spec((_D, _D)),
            _full_spec((1, _D)),
            _full_spec((1, _D)),
            _full_spec((1, 1)),
        ],
        out_specs=(p1_rows, p2_rows, p2_rows, p2_rows),
        out_shape=(
            jax.ShapeDtypeStruct((_N, _D), jnp.float32),
            jax.ShapeDtypeStruct((_N, _D), jnp.float32),
            jax.ShapeDtypeStruct((_N, _D), jnp.float32),
            jax.ShapeDtypeStruct((_N, _D), jnp.float32),
        ),
        scratch_shapes=[pltpu.VMEM((_SPAD, _D), jnp.bfloat16)],
    )(adj, x_bf, W1, b1r, W2, b2r, x, Wm1, bm1r, Wm2, bm2r, wattr, battr)

    return (g_1, g_2, mlp_feat, gk)
